# baseline (speedup 1.0000x reference)
import jax
import jax.numpy as jnp
from jax import lax
from jax.experimental import pallas as pl
from jax.experimental.pallas import tpu as pltpu

_VMEM_LIMIT = 32 * 1024 * 1024
_NODE_TILE = 4096
_TILE_R = 128
_LANE = 128


def _cdiv(a, b):
    return -(-a // b)


def _round_up(v, m):
    return ((v + m - 1) // m) * m


def _node_kernel(x_ref, wdT_ref, wsd_ref, a_ref, b_ref):
    w2t = jnp.dot(wdT_ref[...], wsd_ref[...], preferred_element_type=jnp.float32)
    ab = jnp.dot(x_ref[...], w2t, preferred_element_type=jnp.float32)
    a_ref[...] = ab[:, 0:1]
    b_ref[...] = ab[:, 1:2]


def _edge_kernel(hiA_ref, hiB_ref, loA_ref, loB_ref, t_ref, a_ref, b_ref,
                 score_ref, lsum_ref, msum_ref,
                 hiA_s, hiB_s, rta, rtb, sems):
    cpa = pltpu.make_async_copy(hiA_ref, hiA_s, sems.at[0])
    cpb = pltpu.make_async_copy(hiB_ref, hiB_s, sems.at[1])
    cpa.start()
    cpb.start()
    cpa.wait()
    cpb.wait()

    iota_l = lax.broadcasted_iota(jnp.int32, (_LANE, _LANE), 1)
    ones_col = jnp.ones((_LANE, 1), jnp.float32)

    def body(g, carry):
        lacc, macc = carry
        for mi in range(_LANE):
            rta[pl.ds(mi, 1), :] = a_ref[hiA_s[g, mi]]
            rtb[pl.ds(mi, 1), :] = b_ref[hiB_s[g, mi]]
        lac = loA_ref[g].reshape(_LANE, 1)
        lbc = loB_ref[g].reshape(_LANE, 1)
        m = (jnp.where(iota_l == lac, rta[...], 0.0)
             + jnp.where(iota_l == lbc, rtb[...], 0.0))
        zc = jnp.dot(m, ones_col, preferred_element_type=jnp.float32)
        z = zc.reshape(1, _LANE)
        score = 0.5 * (jnp.tanh(0.5 * z) + 1.0)
        score_ref[g] = score
        p = 0.5 * (jnp.tanh(0.5 * score) + 1.0)
        t = t_ref[g]
        w = jnp.abs(t)
        pt = jnp.where(t > 0.0, p, 1.0 - p)
        om = 1.0 - pt
        lacc = lacc - (om * om) * jnp.log(pt) * w
        macc = macc + jnp.where(t > 0.0, w * (1.0 / 0.75), w * (1.0 / 0.25))
        return lacc, macc

    lacc, macc = lax.fori_loop(
        0, _TILE_R, body,
        (jnp.zeros((1, _LANE), jnp.float32), jnp.zeros((1, _LANE), jnp.float32)))
    lsum_ref[...] = lacc[None]
    msum_ref[...] = macc[None]


def kernel(d_liner_w, f_liner_w, x, edge_index, edge_labels, edge_train_mask):
    alpha = 0.25
    wd = d_liner_w.astype(jnp.float32)
    wf = f_liner_w.astype(jnp.float32)
    c_out, c_in = wd.shape
    w1 = wf[0, 0 * c_out:1 * c_out]
    w2 = wf[0, 1 * c_out:2 * c_out]
    w3 = wf[0, 2 * c_out:3 * c_out]
    wsd = jnp.stack([w1 + w3, w2 - w3], axis=1)
    wdT = wd.T

    n = x.shape[0]
    n_pad = _round_up(n, _NODE_TILE)
    xp = jnp.pad(x.astype(jnp.float32), ((0, n_pad - n), (0, 0)))
    a, b = pl.pallas_call(
        _node_kernel,
        out_shape=(jax.ShapeDtypeStruct((n_pad, 1), jnp.float32),
                   jax.ShapeDtypeStruct((n_pad, 1), jnp.float32)),
        grid=(n_pad // _NODE_TILE,),
        in_specs=[pl.BlockSpec((_NODE_TILE, c_in), lambda i: (i, 0)),
                  pl.BlockSpec((c_in, c_out), lambda i: (0, 0)),
                  pl.BlockSpec((c_out, 2), lambda i: (0, 0))],
        out_specs=(pl.BlockSpec((_NODE_TILE, 1), lambda i: (i, 0)),
                   pl.BlockSpec((_NODE_TILE, 1), lambda i: (i, 0))),
        compiler_params=pltpu.CompilerParams(
            dimension_semantics=("parallel",),
            vmem_limit_bytes=_VMEM_LIMIT),
        cost_estimate=pl.CostEstimate(
            flops=2 * n_pad * c_in * 2 + 2 * c_in * c_out * 2,
            transcendentals=0,
            bytes_accessed=4 * (n_pad * c_in + c_in * c_out + 2 * c_out
                                + 2 * n_pad)),
    )(xp, wdT, wsd)

    n_rows = n_pad // _LANE
    a_tab = a.reshape(n_rows, 1, _LANE)
    b_tab = b.reshape(n_rows, 1, _LANE)

    e = edge_index.shape[1]
    r_pad = _round_up(_cdiv(e, _LANE), _TILE_R)
    e_pad = r_pad * _LANE
    grid_sz = r_pad // _TILE_R

    ei0 = jnp.pad(edge_index[0], (0, e_pad - e))
    ei1 = jnp.pad(edge_index[1], (0, e_pad - e))
    hiA = (ei0 >> 7).reshape(r_pad, _LANE)
    hiB = (ei1 >> 7).reshape(r_pad, _LANE)
    loA = (ei0 & 127).reshape(r_pad, 1, _LANE)
    loB = (ei1 & 127).reshape(r_pad, 1, _LANE)

    lab = edge_labels.astype(jnp.float32)
    mask = edge_train_mask.astype(jnp.float32)
    at = jnp.where(lab > 0.5, 1.0 - alpha, alpha)
    t = jnp.where(lab > 0.5, at, -at) * mask
    t3 = jnp.pad(t, (0, e_pad - e)).reshape(r_pad, 1, _LANE)

    score3, lparts, mparts = pl.pallas_call(
        _edge_kernel,
        out_shape=(jax.ShapeDtypeStruct((r_pad, 1, _LANE), jnp.float32),
                   jax.ShapeDtypeStruct((grid_sz, 1, _LANE), jnp.float32),
                   jax.ShapeDtypeStruct((grid_sz, 1, _LANE), jnp.float32)),
        grid=(grid_sz,),
        in_specs=[pl.BlockSpec((_TILE_R, _LANE), lambda i: (i, 0)),
                  pl.BlockSpec((_TILE_R, _LANE), lambda i: (i, 0)),
                  pl.BlockSpec((_TILE_R, 1, _LANE), lambda i: (i, 0, 0)),
                  pl.BlockSpec((_TILE_R, 1, _LANE), lambda i: (i, 0, 0)),
                  pl.BlockSpec((_TILE_R, 1, _LANE), lambda i: (i, 0, 0)),
                  pl.BlockSpec((n_rows, 1, _LANE), lambda i: (0, 0, 0)),
                  pl.BlockSpec((n_rows, 1, _LANE), lambda i: (0, 0, 0))],
        out_specs=(pl.BlockSpec((_TILE_R, 1, _LANE), lambda i: (i, 0, 0)),
                   pl.BlockSpec((1, 1, _LANE), lambda i: (i, 0, 0)),
                   pl.BlockSpec((1, 1, _LANE), lambda i: (i, 0, 0))),
        scratch_shapes=[pltpu.SMEM((_TILE_R, _LANE), jnp.int32),
                        pltpu.SMEM((_TILE_R, _LANE), jnp.int32),
                        pltpu.VMEM((_LANE, _LANE), jnp.float32),
                        pltpu.VMEM((_LANE, _LANE), jnp.float32),
                        pltpu.SemaphoreType.DMA((2,))],
        compiler_params=pltpu.CompilerParams(
            dimension_semantics=("parallel",),
            vmem_limit_bytes=_VMEM_LIMIT),
        cost_estimate=pl.CostEstimate(
            flops=2 * e_pad * _LANE + 12 * e_pad,
            transcendentals=3 * e_pad,
            bytes_accessed=4 * (e_pad * 4 + 2 * n_pad + e_pad)),
    )(hiA, hiB, loA, loB, t3, a_tab, b_tab)

    edge_score = score3.reshape(-1)[:e]
    edge_loss = jnp.sum(lparts) / jnp.sum(mparts)
    return edge_score, edge_loss


# speedup vs baseline: 1.4129x; 1.4129x over previous
import jax
import jax.numpy as jnp
from jax import lax
from jax.experimental import pallas as pl
from jax.experimental.pallas import tpu as pltpu

_VMEM_LIMIT = 32 * 1024 * 1024
_NODE_TILE = 4096
_TILE_R = 128
_GUNROLL = 4
_LANE = 128


def _cdiv(a, b):
    return -(-a // b)


def _round_up(v, m):
    return ((v + m - 1) // m) * m


def _node_kernel(x_ref, wdT_ref, wsd_ref, abT_ref):
    w2t = jnp.dot(wdT_ref[...], wsd_ref[...], preferred_element_type=jnp.float32)
    abT_ref[...] = lax.dot_general(
        w2t, x_ref[...], (((0,), (1,)), ((), ())),
        preferred_element_type=jnp.float32)


def _edge_kernel(hiA_ref, hiB_ref, loA_ref, loB_ref, t_ref, a_ref, b_ref,
                 score_ref, lsum_ref, msum_ref,
                 hiA_s, hiB_s, rta0, rtb0, rta1, rtb1, rta2, rtb2, rta3, rtb3,
                 sems):
    cpa = pltpu.make_async_copy(hiA_ref, hiA_s, sems.at[0])
    cpb = pltpu.make_async_copy(hiB_ref, hiB_s, sems.at[1])
    cpa.start()
    cpb.start()
    cpa.wait()
    cpb.wait()

    rtas = (rta0, rta1, rta2, rta3)
    rtbs = (rtb0, rtb1, rtb2, rtb3)
    iota_l = lax.broadcasted_iota(jnp.int32, (_LANE, _LANE), 1)
    ones_col = jnp.ones((_LANE, 1), jnp.float32)

    def body(q, carry):
        lacc, macc = carry
        g0 = q * _GUNROLL
        for u in range(_GUNROLL):
            for mi in range(_LANE):
                rtas[u][pl.ds(mi, 1), :] = a_ref[hiA_s[g0 + u, mi]]
                rtbs[u][pl.ds(mi, 1), :] = b_ref[hiB_s[g0 + u, mi]]
        for u in range(_GUNROLL):
            g = g0 + u
            lac = loA_ref[g].reshape(_LANE, 1)
            lbc = loB_ref[g].reshape(_LANE, 1)
            m = (jnp.where(iota_l == lac, rtas[u][...], 0.0)
                 + jnp.where(iota_l == lbc, rtbs[u][...], 0.0))
            zc = jnp.dot(m, ones_col, preferred_element_type=jnp.float32)
            z = zc.reshape(1, _LANE)
            score = 0.5 * (jnp.tanh(0.5 * z) + 1.0)
            score_ref[g] = score
            p = 0.5 * (jnp.tanh(0.5 * score) + 1.0)
            t = t_ref[g]
            w = jnp.abs(t)
            pt = jnp.where(t > 0.0, p, 1.0 - p)
            om = 1.0 - pt
            lacc = lacc - (om * om) * jnp.log(pt) * w
            macc = macc + jnp.where(t > 0.0, w * (1.0 / 0.75), w * (1.0 / 0.25))
        return lacc, macc

    lacc, macc = lax.fori_loop(
        0, _TILE_R // _GUNROLL, body,
        (jnp.zeros((1, _LANE), jnp.float32), jnp.zeros((1, _LANE), jnp.float32)))
    lsum_ref[...] = lacc[None]
    msum_ref[...] = macc[None]


def kernel(d_liner_w, f_liner_w, x, edge_index, edge_labels, edge_train_mask):
    alpha = 0.25
    wd = d_liner_w.astype(jnp.float32)
    wf = f_liner_w.astype(jnp.float32)
    c_out, c_in = wd.shape
    w1 = wf[0, 0 * c_out:1 * c_out]
    w2 = wf[0, 1 * c_out:2 * c_out]
    w3 = wf[0, 2 * c_out:3 * c_out]
    wsd = jnp.stack([w1 + w3, w2 - w3], axis=1)
    wdT = wd.T

    n = x.shape[0]
    n_pad = _round_up(n, _NODE_TILE)
    xp = jnp.pad(x.astype(jnp.float32), ((0, n_pad - n), (0, 0)))
    abT = pl.pallas_call(
        _node_kernel,
        out_shape=jax.ShapeDtypeStruct((2, n_pad), jnp.float32),
        grid=(n_pad // _NODE_TILE,),
        in_specs=[pl.BlockSpec((_NODE_TILE, c_in), lambda i: (i, 0)),
                  pl.BlockSpec((c_in, c_out), lambda i: (0, 0)),
                  pl.BlockSpec((c_out, 2), lambda i: (0, 0))],
        out_specs=pl.BlockSpec((2, _NODE_TILE), lambda i: (0, i)),
        compiler_params=pltpu.CompilerParams(
            dimension_semantics=("parallel",),
            vmem_limit_bytes=_VMEM_LIMIT),
        cost_estimate=pl.CostEstimate(
            flops=2 * n_pad * c_in * 2 + 2 * c_in * c_out * 2,
            transcendentals=0,
            bytes_accessed=4 * (n_pad * c_in + c_in * c_out + 2 * c_out
                                + 2 * n_pad)),
    )(xp, wdT, wsd)

    n_rows = n_pad // _LANE
    a_tab = abT[0].reshape(n_rows, 1, _LANE)
    b_tab = abT[1].reshape(n_rows, 1, _LANE)

    e = edge_index.shape[1]
    r_pad = _round_up(_cdiv(e, _LANE), _TILE_R)
    e_pad = r_pad * _LANE
    grid_sz = r_pad // _TILE_R

    ei0 = jnp.pad(edge_index[0], (0, e_pad - e))
    ei1 = jnp.pad(edge_index[1], (0, e_pad - e))
    hiA = (ei0 >> 7).reshape(r_pad, _LANE)
    hiB = (ei1 >> 7).reshape(r_pad, _LANE)
    loA = (ei0 & 127).reshape(r_pad, 1, _LANE)
    loB = (ei1 & 127).reshape(r_pad, 1, _LANE)

    lab = edge_labels.astype(jnp.float32)
    mask = edge_train_mask.astype(jnp.float32)
    at = jnp.where(lab > 0.5, 1.0 - alpha, alpha)
    t = jnp.where(lab > 0.5, at, -at) * mask
    t3 = jnp.pad(t, (0, e_pad - e)).reshape(r_pad, 1, _LANE)

    score3, lparts, mparts = pl.pallas_call(
        _edge_kernel,
        out_shape=(jax.ShapeDtypeStruct((r_pad, 1, _LANE), jnp.float32),
                   jax.ShapeDtypeStruct((grid_sz, 1, _LANE), jnp.float32),
                   jax.ShapeDtypeStruct((grid_sz, 1, _LANE), jnp.float32)),
        grid=(grid_sz,),
        in_specs=[pl.BlockSpec((_TILE_R, _LANE), lambda i: (i, 0)),
                  pl.BlockSpec((_TILE_R, _LANE), lambda i: (i, 0)),
                  pl.BlockSpec((_TILE_R, 1, _LANE), lambda i: (i, 0, 0)),
                  pl.BlockSpec((_TILE_R, 1, _LANE), lambda i: (i, 0, 0)),
                  pl.BlockSpec((_TILE_R, 1, _LANE), lambda i: (i, 0, 0)),
                  pl.BlockSpec((n_rows, 1, _LANE), lambda i: (0, 0, 0)),
                  pl.BlockSpec((n_rows, 1, _LANE), lambda i: (0, 0, 0))],
        out_specs=(pl.BlockSpec((_TILE_R, 1, _LANE), lambda i: (i, 0, 0)),
                   pl.BlockSpec((1, 1, _LANE), lambda i: (i, 0, 0)),
                   pl.BlockSpec((1, 1, _LANE), lambda i: (i, 0, 0))),
        scratch_shapes=[pltpu.SMEM((_TILE_R, _LANE), jnp.int32),
                        pltpu.SMEM((_TILE_R, _LANE), jnp.int32)]
                       + [pltpu.VMEM((_LANE, _LANE), jnp.float32)] * 8
                       + [pltpu.SemaphoreType.DMA((2,))],
        compiler_params=pltpu.CompilerParams(
            dimension_semantics=("parallel",),
            vmem_limit_bytes=_VMEM_LIMIT),
        cost_estimate=pl.CostEstimate(
            flops=2 * e_pad * _LANE + 12 * e_pad,
            transcendentals=3 * e_pad,
            bytes_accessed=4 * (e_pad * 4 + 2 * n_pad + e_pad)),
    )(hiA, hiB, loA, loB, t3, a_tab, b_tab)

    edge_score = score3.reshape(-1)[:e]
    edge_loss = jnp.sum(lparts) / jnp.sum(mparts)
    return edge_score, edge_loss


# speedup vs baseline: 1.4208x; 1.0056x over previous
import jax
import jax.numpy as jnp
from jax import lax
from jax.experimental import pallas as pl
from jax.experimental.pallas import tpu as pltpu

_VMEM_LIMIT = 32 * 1024 * 1024
_NODE_TILE = 4096
_TILE_R = 256
_GUNROLL = 4
_LANE = 128


def _cdiv(a, b):
    return -(-a // b)


def _round_up(v, m):
    return ((v + m - 1) // m) * m


def _node_kernel(x_ref, wdT_ref, wsd_ref, abT_ref):
    w2t = jnp.dot(wdT_ref[...], wsd_ref[...], preferred_element_type=jnp.float32)
    abT_ref[...] = lax.dot_general(
        w2t, x_ref[...], (((0,), (1,)), ((), ())),
        preferred_element_type=jnp.float32)


def _edge_kernel(hiA_ref, hiB_ref, loA_ref, loB_ref, t_ref, a_ref, b_ref,
                 score_ref, lsum_ref, msum_ref,
                 hiA_s, hiB_s, rta0, rtb0, rta1, rtb1, rta2, rtb2, rta3, rtb3,
                 sems):
    cpa = pltpu.make_async_copy(hiA_ref, hiA_s, sems.at[0])
    cpb = pltpu.make_async_copy(hiB_ref, hiB_s, sems.at[1])
    cpa.start()
    cpb.start()
    cpa.wait()
    cpb.wait()

    rtas = (rta0, rta1, rta2, rta3)
    rtbs = (rtb0, rtb1, rtb2, rtb3)
    iota_l = lax.broadcasted_iota(jnp.int32, (_LANE, _LANE), 1)
    ones_col = jnp.ones((_LANE, 1), jnp.float32)

    def body(q, carry):
        lacc, macc = carry
        g0 = q * _GUNROLL
        for u in range(_GUNROLL):
            for mi in range(_LANE):
                rtas[u][pl.ds(mi, 1), :] = a_ref[hiA_s[g0 + u, mi]]
                rtbs[u][pl.ds(mi, 1), :] = b_ref[hiB_s[g0 + u, mi]]
        for u in range(_GUNROLL):
            g = g0 + u
            lac = loA_ref[g].reshape(_LANE, 1)
            lbc = loB_ref[g].reshape(_LANE, 1)
            m = (jnp.where(iota_l == lac, rtas[u][...], 0.0)
                 + jnp.where(iota_l == lbc, rtbs[u][...], 0.0))
            zc = jnp.dot(m, ones_col, preferred_element_type=jnp.float32)
            z = zc.reshape(1, _LANE)
            score = 0.5 * (jnp.tanh(0.5 * z) + 1.0)
            score_ref[g] = score
            p = 0.5 * (jnp.tanh(0.5 * score) + 1.0)
            t = t_ref[g]
            w = jnp.abs(t)
            pt = jnp.where(t > 0.0, p, 1.0 - p)
            om = 1.0 - pt
            lacc = lacc - (om * om) * jnp.log(pt) * w
            macc = macc + jnp.where(t > 0.0, w * (1.0 / 0.75), w * (1.0 / 0.25))
        return lacc, macc

    lacc, macc = lax.fori_loop(
        0, _TILE_R // _GUNROLL, body,
        (jnp.zeros((1, _LANE), jnp.float32), jnp.zeros((1, _LANE), jnp.float32)))
    lsum_ref[...] = lacc[None]
    msum_ref[...] = macc[None]


def kernel(d_liner_w, f_liner_w, x, edge_index, edge_labels, edge_train_mask):
    alpha = 0.25
    wd = d_liner_w.astype(jnp.float32)
    wf = f_liner_w.astype(jnp.float32)
    c_out, c_in = wd.shape
    w1 = wf[0, 0 * c_out:1 * c_out]
    w2 = wf[0, 1 * c_out:2 * c_out]
    w3 = wf[0, 2 * c_out:3 * c_out]
    wsd = jnp.stack([w1 + w3, w2 - w3], axis=1)
    wdT = wd.T

    n = x.shape[0]
    n_pad = _round_up(n, _NODE_TILE)
    xp = jnp.pad(x.astype(jnp.float32), ((0, n_pad - n), (0, 0)))
    abT = pl.pallas_call(
        _node_kernel,
        out_shape=jax.ShapeDtypeStruct((2, n_pad), jnp.float32),
        grid=(n_pad // _NODE_TILE,),
        in_specs=[pl.BlockSpec((_NODE_TILE, c_in), lambda i: (i, 0)),
                  pl.BlockSpec((c_in, c_out), lambda i: (0, 0)),
                  pl.BlockSpec((c_out, 2), lambda i: (0, 0))],
        out_specs=pl.BlockSpec((2, _NODE_TILE), lambda i: (0, i)),
        compiler_params=pltpu.CompilerParams(
            dimension_semantics=("parallel",),
            vmem_limit_bytes=_VMEM_LIMIT),
        cost_estimate=pl.CostEstimate(
            flops=2 * n_pad * c_in * 2 + 2 * c_in * c_out * 2,
            transcendentals=0,
            bytes_accessed=4 * (n_pad * c_in + c_in * c_out + 2 * c_out
                                + 2 * n_pad)),
    )(xp, wdT, wsd)

    n_rows = n_pad // _LANE
    a_tab = abT[0].reshape(n_rows, 1, _LANE)
    b_tab = abT[1].reshape(n_rows, 1, _LANE)

    e = edge_index.shape[1]
    r_pad = _round_up(_cdiv(e, _LANE), _TILE_R)
    e_pad = r_pad * _LANE
    grid_sz = r_pad // _TILE_R

    ei0 = jnp.pad(edge_index[0], (0, e_pad - e))
    ei1 = jnp.pad(edge_index[1], (0, e_pad - e))
    hiA = (ei0 >> 7).reshape(r_pad, _LANE)
    hiB = (ei1 >> 7).reshape(r_pad, _LANE)
    loA = (ei0 & 127).reshape(r_pad, 1, _LANE)
    loB = (ei1 & 127).reshape(r_pad, 1, _LANE)

    lab = edge_labels.astype(jnp.float32)
    mask = edge_train_mask.astype(jnp.float32)
    at = jnp.where(lab > 0.5, 1.0 - alpha, alpha)
    t = jnp.where(lab > 0.5, at, -at) * mask
    t3 = jnp.pad(t, (0, e_pad - e)).reshape(r_pad, 1, _LANE)

    score3, lparts, mparts = pl.pallas_call(
        _edge_kernel,
        out_shape=(jax.ShapeDtypeStruct((r_pad, 1, _LANE), jnp.float32),
                   jax.ShapeDtypeStruct((grid_sz, 1, _LANE), jnp.float32),
                   jax.ShapeDtypeStruct((grid_sz, 1, _LANE), jnp.float32)),
        grid=(grid_sz,),
        in_specs=[pl.BlockSpec((_TILE_R, _LANE), lambda i: (i, 0)),
                  pl.BlockSpec((_TILE_R, _LANE), lambda i: (i, 0)),
                  pl.BlockSpec((_TILE_R, 1, _LANE), lambda i: (i, 0, 0)),
                  pl.BlockSpec((_TILE_R, 1, _LANE), lambda i: (i, 0, 0)),
                  pl.BlockSpec((_TILE_R, 1, _LANE), lambda i: (i, 0, 0)),
                  pl.BlockSpec((n_rows, 1, _LANE), lambda i: (0, 0, 0)),
                  pl.BlockSpec((n_rows, 1, _LANE), lambda i: (0, 0, 0))],
        out_specs=(pl.BlockSpec((_TILE_R, 1, _LANE), lambda i: (i, 0, 0)),
                   pl.BlockSpec((1, 1, _LANE), lambda i: (i, 0, 0)),
                   pl.BlockSpec((1, 1, _LANE), lambda i: (i, 0, 0))),
        scratch_shapes=[pltpu.SMEM((_TILE_R, _LANE), jnp.int32),
                        pltpu.SMEM((_TILE_R, _LANE), jnp.int32)]
                       + [pltpu.VMEM((_LANE, _LANE), jnp.float32)] * 8
                       + [pltpu.SemaphoreType.DMA((2,))],
        compiler_params=pltpu.CompilerParams(
            dimension_semantics=("parallel",),
            vmem_limit_bytes=_VMEM_LIMIT),
        cost_estimate=pl.CostEstimate(
            flops=2 * e_pad * _LANE + 12 * e_pad,
            transcendentals=3 * e_pad,
            bytes_accessed=4 * (e_pad * 4 + 2 * n_pad + e_pad)),
    )(hiA, hiB, loA, loB, t3, a_tab, b_tab)

    edge_score = score3.reshape(-1)[:e]
    edge_loss = jnp.sum(lparts) / jnp.sum(mparts)
    return edge_score, edge_loss


# speedup vs baseline: 1.6609x; 1.1690x over previous
import jax
import jax.numpy as jnp
from jax import lax
from jax.experimental import pallas as pl
from jax.experimental.pallas import tpu as pltpu

_VMEM_LIMIT = 32 * 1024 * 1024
_NODE_TILE = 4096
_TILE_R = 256
_GUNROLL = 4
_LANE = 128


def _cdiv(a, b):
    return -(-a // b)


def _round_up(v, m):
    return ((v + m - 1) // m) * m


def _node_kernel(x_ref, wdT_ref, wsd_ref, abT_ref):
    w2t = jnp.dot(wdT_ref[...], wsd_ref[...], preferred_element_type=jnp.float32)
    abT_ref[...] = lax.dot_general(
        w2t, x_ref[...], (((0,), (1,)), ((), ())),
        preferred_element_type=jnp.float32)


def _edge_kernel(hiA_ref, hiB_ref, loA_ref, loB_ref, t_ref, a_ref, b_ref,
                 score_ref, lsum_ref, msum_ref,
                 hiA_s, hiB_s, rta0, rtb0, rta1, rtb1, rta2, rtb2, rta3, rtb3,
                 sems):
    cpa = pltpu.make_async_copy(hiA_ref, hiA_s, sems.at[0])
    cpb = pltpu.make_async_copy(hiB_ref, hiB_s, sems.at[1])
    cpa.start()
    cpb.start()
    cpa.wait()
    cpb.wait()

    rtas = (rta0, rta1, rta2, rta3)
    rtbs = (rtb0, rtb1, rtb2, rtb3)
    iota_l = lax.broadcasted_iota(jnp.int32, (_LANE, _LANE), 1)
    ones_col = jnp.ones((_LANE, 1), jnp.float32)

    def body(q, carry):
        lacc, macc = carry
        g0 = q * _GUNROLL
        for u in range(_GUNROLL):
            for mi in range(_LANE):
                rtas[u][pl.ds(mi, 1), :] = a_ref[hiA_s[g0 + u, mi]]
                rtbs[u][pl.ds(mi, 1), :] = b_ref[hiB_s[g0 + u, mi]]
        zcs = []
        for u in range(_GUNROLL):
            g = g0 + u
            lac = loA_ref[g].reshape(_LANE, 1)
            lbc = loB_ref[g].reshape(_LANE, 1)
            m = (jnp.where(iota_l == lac, rtas[u][...], 0.0)
                 + jnp.where(iota_l == lbc, rtbs[u][...], 0.0))
            zcs.append(jnp.dot(m, ones_col, preferred_element_type=jnp.float32))
        z = jnp.concatenate(zcs, axis=1).T
        score = 0.5 * (jnp.tanh(0.5 * z) + 1.0)
        score_ref[q] = score
        p = 0.5 * (jnp.tanh(0.5 * score) + 1.0)
        t = t_ref[q]
        w = jnp.abs(t)
        pt = jnp.where(t > 0.0, p, 1.0 - p)
        om = 1.0 - pt
        lacc = lacc - (om * om) * jnp.log(pt) * w
        macc = macc + jnp.where(t > 0.0, w * (1.0 / 0.75), w * (1.0 / 0.25))
        return lacc, macc

    lacc, macc = lax.fori_loop(
        0, _TILE_R // _GUNROLL, body,
        (jnp.zeros((_GUNROLL, _LANE), jnp.float32),
         jnp.zeros((_GUNROLL, _LANE), jnp.float32)))
    lsum_ref[...] = jnp.sum(lacc, axis=0, keepdims=True)[None]
    msum_ref[...] = jnp.sum(macc, axis=0, keepdims=True)[None]


def kernel(d_liner_w, f_liner_w, x, edge_index, edge_labels, edge_train_mask):
    alpha = 0.25
    wd = d_liner_w.astype(jnp.float32)
    wf = f_liner_w.astype(jnp.float32)
    c_out, c_in = wd.shape
    w1 = wf[0, 0 * c_out:1 * c_out]
    w2 = wf[0, 1 * c_out:2 * c_out]
    w3 = wf[0, 2 * c_out:3 * c_out]
    wsd = jnp.stack([w1 + w3, w2 - w3], axis=1)
    wdT = wd.T

    n = x.shape[0]
    n_pad = _round_up(n, _NODE_TILE)
    xp = jnp.pad(x.astype(jnp.float32), ((0, n_pad - n), (0, 0)))
    abT = pl.pallas_call(
        _node_kernel,
        out_shape=jax.ShapeDtypeStruct((2, n_pad), jnp.float32),
        grid=(n_pad // _NODE_TILE,),
        in_specs=[pl.BlockSpec((_NODE_TILE, c_in), lambda i: (i, 0)),
                  pl.BlockSpec((c_in, c_out), lambda i: (0, 0)),
                  pl.BlockSpec((c_out, 2), lambda i: (0, 0))],
        out_specs=pl.BlockSpec((2, _NODE_TILE), lambda i: (0, i)),
        compiler_params=pltpu.CompilerParams(
            dimension_semantics=("parallel",),
            vmem_limit_bytes=_VMEM_LIMIT),
        cost_estimate=pl.CostEstimate(
            flops=2 * n_pad * c_in * 2 + 2 * c_in * c_out * 2,
            transcendentals=0,
            bytes_accessed=4 * (n_pad * c_in + c_in * c_out + 2 * c_out
                                + 2 * n_pad)),
    )(xp, wdT, wsd)

    n_rows = n_pad // _LANE
    a_tab = abT[0].reshape(n_rows, 1, _LANE)
    b_tab = abT[1].reshape(n_rows, 1, _LANE)

    e = edge_index.shape[1]
    r_pad = _round_up(_cdiv(e, _LANE), _TILE_R)
    e_pad = r_pad * _LANE
    grid_sz = r_pad // _TILE_R

    ei0 = jnp.pad(edge_index[0], (0, e_pad - e))
    ei1 = jnp.pad(edge_index[1], (0, e_pad - e))
    hiA = (ei0 >> 7).reshape(r_pad, _LANE)
    hiB = (ei1 >> 7).reshape(r_pad, _LANE)
    loA = (ei0 & 127).reshape(r_pad, 1, _LANE)
    loB = (ei1 & 127).reshape(r_pad, 1, _LANE)

    lab = edge_labels.astype(jnp.float32)
    mask = edge_train_mask.astype(jnp.float32)
    at = jnp.where(lab > 0.5, 1.0 - alpha, alpha)
    t = jnp.where(lab > 0.5, at, -at) * mask
    t3 = jnp.pad(t, (0, e_pad - e)).reshape(r_pad // _GUNROLL, _GUNROLL, _LANE)

    score3, lparts, mparts = pl.pallas_call(
        _edge_kernel,
        out_shape=(jax.ShapeDtypeStruct((r_pad // _GUNROLL, _GUNROLL, _LANE),
                                        jnp.float32),
                   jax.ShapeDtypeStruct((grid_sz, 1, _LANE), jnp.float32),
                   jax.ShapeDtypeStruct((grid_sz, 1, _LANE), jnp.float32)),
        grid=(grid_sz,),
        in_specs=[pl.BlockSpec((_TILE_R, _LANE), lambda i: (i, 0)),
                  pl.BlockSpec((_TILE_R, _LANE), lambda i: (i, 0)),
                  pl.BlockSpec((_TILE_R, 1, _LANE), lambda i: (i, 0, 0)),
                  pl.BlockSpec((_TILE_R, 1, _LANE), lambda i: (i, 0, 0)),
                  pl.BlockSpec((_TILE_R // _GUNROLL, _GUNROLL, _LANE),
                               lambda i: (i, 0, 0)),
                  pl.BlockSpec((n_rows, 1, _LANE), lambda i: (0, 0, 0)),
                  pl.BlockSpec((n_rows, 1, _LANE), lambda i: (0, 0, 0))],
        out_specs=(pl.BlockSpec((_TILE_R // _GUNROLL, _GUNROLL, _LANE),
                                lambda i: (i, 0, 0)),
                   pl.BlockSpec((1, 1, _LANE), lambda i: (i, 0, 0)),
                   pl.BlockSpec((1, 1, _LANE), lambda i: (i, 0, 0))),
        scratch_shapes=[pltpu.SMEM((_TILE_R, _LANE), jnp.int32),
                        pltpu.SMEM((_TILE_R, _LANE), jnp.int32)]
                       + [pltpu.VMEM((_LANE, _LANE), jnp.float32)] * 8
                       + [pltpu.SemaphoreType.DMA((2,))],
        compiler_params=pltpu.CompilerParams(
            dimension_semantics=("parallel",),
            vmem_limit_bytes=_VMEM_LIMIT),
        cost_estimate=pl.CostEstimate(
            flops=2 * e_pad * _LANE + 12 * e_pad,
            transcendentals=3 * e_pad,
            bytes_accessed=4 * (e_pad * 4 + 2 * n_pad + e_pad)),
    )(hiA, hiB, loA, loB, t3, a_tab, b_tab)

    edge_score = score3.reshape(-1)[:e]
    edge_loss = jnp.sum(lparts) / jnp.sum(mparts)
    return edge_score, edge_loss


# speedup vs baseline: 1.8108x; 1.0902x over previous
import jax
import jax.numpy as jnp
from jax import lax
from jax.experimental import pallas as pl
from jax.experimental.pallas import tpu as pltpu

_VMEM_LIMIT = 32 * 1024 * 1024
_NODE_TILE = 4096
_TILE_R = 256
_GUNROLL = 8
_LANE = 128


def _cdiv(a, b):
    return -(-a // b)


def _round_up(v, m):
    return ((v + m - 1) // m) * m


def _node_kernel(x_ref, wdT_ref, wsd_ref, abT_ref):
    w2t = jnp.dot(wdT_ref[...], wsd_ref[...], preferred_element_type=jnp.float32)
    abT_ref[...] = lax.dot_general(
        w2t, x_ref[...], (((0,), (1,)), ((), ())),
        preferred_element_type=jnp.float32)


def _edge_kernel(hiA_ref, hiB_ref, loA_ref, loB_ref, t_ref, a_ref, b_ref,
                 score_ref, lsum_ref, msum_ref,
                 hiA_s, hiB_s, rta0, rtb0, rta1, rtb1, rta2, rtb2, rta3, rtb3,
                 rta4, rtb4, rta5, rtb5, rta6, rtb6, rta7, rtb7,
                 sems):
    cpa = pltpu.make_async_copy(hiA_ref, hiA_s, sems.at[0])
    cpb = pltpu.make_async_copy(hiB_ref, hiB_s, sems.at[1])
    cpa.start()
    cpb.start()
    cpa.wait()
    cpb.wait()

    rtas = (rta0, rta1, rta2, rta3, rta4, rta5, rta6, rta7)
    rtbs = (rtb0, rtb1, rtb2, rtb3, rtb4, rtb5, rtb6, rtb7)
    iota_l = lax.broadcasted_iota(jnp.int32, (_LANE, _LANE), 1)
    ones_col = jnp.ones((_LANE, 1), jnp.float32)

    def body(q, carry):
        lacc, macc = carry
        g0 = q * _GUNROLL
        for u in range(_GUNROLL):
            for mi in range(_LANE):
                rtas[u][pl.ds(mi, 1), :] = a_ref[hiA_s[g0 + u, mi]]
                rtbs[u][pl.ds(mi, 1), :] = b_ref[hiB_s[g0 + u, mi]]
        zcs = []
        for u in range(_GUNROLL):
            g = g0 + u
            lac = loA_ref[g].reshape(_LANE, 1)
            lbc = loB_ref[g].reshape(_LANE, 1)
            m = (jnp.where(iota_l == lac, rtas[u][...], 0.0)
                 + jnp.where(iota_l == lbc, rtbs[u][...], 0.0))
            zcs.append(jnp.dot(m, ones_col, preferred_element_type=jnp.float32))
        z = jnp.concatenate(zcs, axis=1).T
        score = 0.5 * (jnp.tanh(0.5 * z) + 1.0)
        score_ref[q] = score
        p = 0.5 * (jnp.tanh(0.5 * score) + 1.0)
        t = t_ref[q]
        w = jnp.abs(t)
        pt = jnp.where(t > 0.0, p, 1.0 - p)
        om = 1.0 - pt
        lacc = lacc - (om * om) * jnp.log(pt) * w
        macc = macc + jnp.where(t > 0.0, w * (1.0 / 0.75), w * (1.0 / 0.25))
        return lacc, macc

    lacc, macc = lax.fori_loop(
        0, _TILE_R // _GUNROLL, body,
        (jnp.zeros((_GUNROLL, _LANE), jnp.float32),
         jnp.zeros((_GUNROLL, _LANE), jnp.float32)))
    lsum_ref[...] = jnp.sum(lacc, axis=0, keepdims=True)[None]
    msum_ref[...] = jnp.sum(macc, axis=0, keepdims=True)[None]


def kernel(d_liner_w, f_liner_w, x, edge_index, edge_labels, edge_train_mask):
    alpha = 0.25
    wd = d_liner_w.astype(jnp.float32)
    wf = f_liner_w.astype(jnp.float32)
    c_out, c_in = wd.shape
    w1 = wf[0, 0 * c_out:1 * c_out]
    w2 = wf[0, 1 * c_out:2 * c_out]
    w3 = wf[0, 2 * c_out:3 * c_out]
    wsd = jnp.stack([w1 + w3, w2 - w3], axis=1)
    wdT = wd.T

    n = x.shape[0]
    n_pad = _round_up(n, _NODE_TILE)
    xp = jnp.pad(x.astype(jnp.float32), ((0, n_pad - n), (0, 0)))
    abT = pl.pallas_call(
        _node_kernel,
        out_shape=jax.ShapeDtypeStruct((2, n_pad), jnp.float32),
        grid=(n_pad // _NODE_TILE,),
        in_specs=[pl.BlockSpec((_NODE_TILE, c_in), lambda i: (i, 0)),
                  pl.BlockSpec((c_in, c_out), lambda i: (0, 0)),
                  pl.BlockSpec((c_out, 2), lambda i: (0, 0))],
        out_specs=pl.BlockSpec((2, _NODE_TILE), lambda i: (0, i)),
        compiler_params=pltpu.CompilerParams(
            dimension_semantics=("parallel",),
            vmem_limit_bytes=_VMEM_LIMIT),
        cost_estimate=pl.CostEstimate(
            flops=2 * n_pad * c_in * 2 + 2 * c_in * c_out * 2,
            transcendentals=0,
            bytes_accessed=4 * (n_pad * c_in + c_in * c_out + 2 * c_out
                                + 2 * n_pad)),
    )(xp, wdT, wsd)

    n_rows = n_pad // _LANE
    a_tab = abT[0].reshape(n_rows, 1, _LANE)
    b_tab = abT[1].reshape(n_rows, 1, _LANE)

    e = edge_index.shape[1]
    r_pad = _round_up(_cdiv(e, _LANE), _TILE_R)
    e_pad = r_pad * _LANE
    grid_sz = r_pad // _TILE_R

    ei0 = jnp.pad(edge_index[0], (0, e_pad - e))
    ei1 = jnp.pad(edge_index[1], (0, e_pad - e))
    hiA = (ei0 >> 7).reshape(r_pad, _LANE)
    hiB = (ei1 >> 7).reshape(r_pad, _LANE)
    loA = (ei0 & 127).reshape(r_pad, 1, _LANE)
    loB = (ei1 & 127).reshape(r_pad, 1, _LANE)

    lab = edge_labels.astype(jnp.float32)
    mask = edge_train_mask.astype(jnp.float32)
    at = jnp.where(lab > 0.5, 1.0 - alpha, alpha)
    t = jnp.where(lab > 0.5, at, -at) * mask
    t3 = jnp.pad(t, (0, e_pad - e)).reshape(r_pad // _GUNROLL, _GUNROLL, _LANE)

    score3, lparts, mparts = pl.pallas_call(
        _edge_kernel,
        out_shape=(jax.ShapeDtypeStruct((r_pad // _GUNROLL, _GUNROLL, _LANE),
                                        jnp.float32),
                   jax.ShapeDtypeStruct((grid_sz, 1, _LANE), jnp.float32),
                   jax.ShapeDtypeStruct((grid_sz, 1, _LANE), jnp.float32)),
        grid=(grid_sz,),
        in_specs=[pl.BlockSpec((_TILE_R, _LANE), lambda i: (i, 0)),
                  pl.BlockSpec((_TILE_R, _LANE), lambda i: (i, 0)),
                  pl.BlockSpec((_TILE_R, 1, _LANE), lambda i: (i, 0, 0)),
                  pl.BlockSpec((_TILE_R, 1, _LANE), lambda i: (i, 0, 0)),
                  pl.BlockSpec((_TILE_R // _GUNROLL, _GUNROLL, _LANE),
                               lambda i: (i, 0, 0)),
                  pl.BlockSpec((n_rows, 1, _LANE), lambda i: (0, 0, 0)),
                  pl.BlockSpec((n_rows, 1, _LANE), lambda i: (0, 0, 0))],
        out_specs=(pl.BlockSpec((_TILE_R // _GUNROLL, _GUNROLL, _LANE),
                                lambda i: (i, 0, 0)),
                   pl.BlockSpec((1, 1, _LANE), lambda i: (i, 0, 0)),
                   pl.BlockSpec((1, 1, _LANE), lambda i: (i, 0, 0))),
        scratch_shapes=[pltpu.SMEM((_TILE_R, _LANE), jnp.int32),
                        pltpu.SMEM((_TILE_R, _LANE), jnp.int32)]
                       + [pltpu.VMEM((_LANE, _LANE), jnp.float32)] * 16
                       + [pltpu.SemaphoreType.DMA((2,))],
        compiler_params=pltpu.CompilerParams(
            dimension_semantics=("parallel",),
            vmem_limit_bytes=_VMEM_LIMIT),
        cost_estimate=pl.CostEstimate(
            flops=2 * e_pad * _LANE + 12 * e_pad,
            transcendentals=3 * e_pad,
            bytes_accessed=4 * (e_pad * 4 + 2 * n_pad + e_pad)),
    )(hiA, hiB, loA, loB, t3, a_tab, b_tab)

    edge_score = score3.reshape(-1)[:e]
    edge_loss = jnp.sum(lparts) / jnp.sum(mparts)
    return edge_score, edge_loss


# speedup vs baseline: 1.9339x; 1.0680x over previous
import jax
import jax.numpy as jnp
from jax import lax
from jax.experimental import pallas as pl
from jax.experimental.pallas import tpu as pltpu

_VMEM_LIMIT = 32 * 1024 * 1024
_NODE_TILE = 4096
_TILE_R = 256
_GUNROLL = 16
_LANE = 128


def _cdiv(a, b):
    return -(-a // b)


def _round_up(v, m):
    return ((v + m - 1) // m) * m


def _node_kernel(x_ref, wdT_ref, wsd_ref, abT_ref):
    w2t = jnp.dot(wdT_ref[...], wsd_ref[...], preferred_element_type=jnp.float32)
    abT_ref[...] = lax.dot_general(
        w2t, x_ref[...], (((0,), (1,)), ((), ())),
        preferred_element_type=jnp.float32)


def _edge_kernel(hiA_ref, hiB_ref, loA_ref, loB_ref, t_ref, a_ref, b_ref,
                 score_ref, lsum_ref, msum_ref,
                 hiA_s, hiB_s, rta0, rtb0, rta1, rtb1, rta2, rtb2, rta3, rtb3,
                 rta4, rtb4, rta5, rtb5, rta6, rtb6, rta7, rtb7,
                 rta8, rtb8, rta9, rtb9, rta10, rtb10, rta11, rtb11,
                 rta12, rtb12, rta13, rtb13, rta14, rtb14, rta15, rtb15,
                 sems):
    cpa = pltpu.make_async_copy(hiA_ref, hiA_s, sems.at[0])
    cpb = pltpu.make_async_copy(hiB_ref, hiB_s, sems.at[1])
    cpa.start()
    cpb.start()
    cpa.wait()
    cpb.wait()

    rtas = (rta0, rta1, rta2, rta3, rta4, rta5, rta6, rta7,
            rta8, rta9, rta10, rta11, rta12, rta13, rta14, rta15)
    rtbs = (rtb0, rtb1, rtb2, rtb3, rtb4, rtb5, rtb6, rtb7,
            rtb8, rtb9, rtb10, rtb11, rtb12, rtb13, rtb14, rtb15)
    iota_l = lax.broadcasted_iota(jnp.int32, (_LANE, _LANE), 1)
    ones_col = jnp.ones((_LANE, 1), jnp.float32)

    def body(q, carry):
        lacc, macc = carry
        g0 = q * _GUNROLL
        for u in range(_GUNROLL):
            for mi in range(_LANE):
                rtas[u][pl.ds(mi, 1), :] = a_ref[hiA_s[g0 + u, mi]]
                rtbs[u][pl.ds(mi, 1), :] = b_ref[hiB_s[g0 + u, mi]]
        zcs = []
        for u in range(_GUNROLL):
            g = g0 + u
            lac = loA_ref[g].reshape(_LANE, 1)
            lbc = loB_ref[g].reshape(_LANE, 1)
            m = (jnp.where(iota_l == lac, rtas[u][...], 0.0)
                 + jnp.where(iota_l == lbc, rtbs[u][...], 0.0))
            zcs.append(jnp.dot(m, ones_col, preferred_element_type=jnp.float32))
        z = jnp.concatenate(zcs, axis=1).T
        score = 0.5 * (jnp.tanh(0.5 * z) + 1.0)
        score_ref[q] = score
        p = 0.5 * (jnp.tanh(0.5 * score) + 1.0)
        t = t_ref[q]
        w = jnp.abs(t)
        pt = jnp.where(t > 0.0, p, 1.0 - p)
        om = 1.0 - pt
        lacc = lacc - (om * om) * jnp.log(pt) * w
        macc = macc + jnp.where(t > 0.0, w * (1.0 / 0.75), w * (1.0 / 0.25))
        return lacc, macc

    lacc, macc = lax.fori_loop(
        0, _TILE_R // _GUNROLL, body,
        (jnp.zeros((_GUNROLL, _LANE), jnp.float32),
         jnp.zeros((_GUNROLL, _LANE), jnp.float32)))
    lsum_ref[...] = jnp.sum(lacc, axis=0, keepdims=True)[None]
    msum_ref[...] = jnp.sum(macc, axis=0, keepdims=True)[None]


def kernel(d_liner_w, f_liner_w, x, edge_index, edge_labels, edge_train_mask):
    alpha = 0.25
    wd = d_liner_w.astype(jnp.float32)
    wf = f_liner_w.astype(jnp.float32)
    c_out, c_in = wd.shape
    w1 = wf[0, 0 * c_out:1 * c_out]
    w2 = wf[0, 1 * c_out:2 * c_out]
    w3 = wf[0, 2 * c_out:3 * c_out]
    wsd = jnp.stack([w1 + w3, w2 - w3], axis=1)
    wdT = wd.T

    n = x.shape[0]
    n_pad = _round_up(n, _NODE_TILE)
    xp = jnp.pad(x.astype(jnp.float32), ((0, n_pad - n), (0, 0)))
    abT = pl.pallas_call(
        _node_kernel,
        out_shape=jax.ShapeDtypeStruct((2, n_pad), jnp.float32),
        grid=(n_pad // _NODE_TILE,),
        in_specs=[pl.BlockSpec((_NODE_TILE, c_in), lambda i: (i, 0)),
                  pl.BlockSpec((c_in, c_out), lambda i: (0, 0)),
                  pl.BlockSpec((c_out, 2), lambda i: (0, 0))],
        out_specs=pl.BlockSpec((2, _NODE_TILE), lambda i: (0, i)),
        compiler_params=pltpu.CompilerParams(
            dimension_semantics=("parallel",),
            vmem_limit_bytes=_VMEM_LIMIT),
        cost_estimate=pl.CostEstimate(
            flops=2 * n_pad * c_in * 2 + 2 * c_in * c_out * 2,
            transcendentals=0,
            bytes_accessed=4 * (n_pad * c_in + c_in * c_out + 2 * c_out
                                + 2 * n_pad)),
    )(xp, wdT, wsd)

    n_rows = n_pad // _LANE
    a_tab = abT[0].reshape(n_rows, 1, _LANE)
    b_tab = abT[1].reshape(n_rows, 1, _LANE)

    e = edge_index.shape[1]
    r_pad = _round_up(_cdiv(e, _LANE), _TILE_R)
    e_pad = r_pad * _LANE
    grid_sz = r_pad // _TILE_R

    ei0 = jnp.pad(edge_index[0], (0, e_pad - e))
    ei1 = jnp.pad(edge_index[1], (0, e_pad - e))
    hiA = (ei0 >> 7).reshape(r_pad, _LANE)
    hiB = (ei1 >> 7).reshape(r_pad, _LANE)
    loA = (ei0 & 127).reshape(r_pad, 1, _LANE)
    loB = (ei1 & 127).reshape(r_pad, 1, _LANE)

    lab = edge_labels.astype(jnp.float32)
    mask = edge_train_mask.astype(jnp.float32)
    at = jnp.where(lab > 0.5, 1.0 - alpha, alpha)
    t = jnp.where(lab > 0.5, at, -at) * mask
    t3 = jnp.pad(t, (0, e_pad - e)).reshape(r_pad // _GUNROLL, _GUNROLL, _LANE)

    score3, lparts, mparts = pl.pallas_call(
        _edge_kernel,
        out_shape=(jax.ShapeDtypeStruct((r_pad // _GUNROLL, _GUNROLL, _LANE),
                                        jnp.float32),
                   jax.ShapeDtypeStruct((grid_sz, 1, _LANE), jnp.float32),
                   jax.ShapeDtypeStruct((grid_sz, 1, _LANE), jnp.float32)),
        grid=(grid_sz,),
        in_specs=[pl.BlockSpec((_TILE_R, _LANE), lambda i: (i, 0)),
                  pl.BlockSpec((_TILE_R, _LANE), lambda i: (i, 0)),
                  pl.BlockSpec((_TILE_R, 1, _LANE), lambda i: (i, 0, 0)),
                  pl.BlockSpec((_TILE_R, 1, _LANE), lambda i: (i, 0, 0)),
                  pl.BlockSpec((_TILE_R // _GUNROLL, _GUNROLL, _LANE),
                               lambda i: (i, 0, 0)),
                  pl.BlockSpec((n_rows, 1, _LANE), lambda i: (0, 0, 0)),
                  pl.BlockSpec((n_rows, 1, _LANE), lambda i: (0, 0, 0))],
        out_specs=(pl.BlockSpec((_TILE_R // _GUNROLL, _GUNROLL, _LANE),
                                lambda i: (i, 0, 0)),
                   pl.BlockSpec((1, 1, _LANE), lambda i: (i, 0, 0)),
                   pl.BlockSpec((1, 1, _LANE), lambda i: (i, 0, 0))),
        scratch_shapes=[pltpu.SMEM((_TILE_R, _LANE), jnp.int32),
                        pltpu.SMEM((_TILE_R, _LANE), jnp.int32)]
                       + [pltpu.VMEM((_LANE, _LANE), jnp.float32)] * 32
                       + [pltpu.SemaphoreType.DMA((2,))],
        compiler_params=pltpu.CompilerParams(
            dimension_semantics=("parallel",),
            vmem_limit_bytes=_VMEM_LIMIT),
        cost_estimate=pl.CostEstimate(
            flops=2 * e_pad * _LANE + 12 * e_pad,
            transcendentals=3 * e_pad,
            bytes_accessed=4 * (e_pad * 4 + 2 * n_pad + e_pad)),
    )(hiA, hiB, loA, loB, t3, a_tab, b_tab)

    edge_score = score3.reshape(-1)[:e]
    edge_loss = jnp.sum(lparts) / jnp.sum(mparts)
    return edge_score, edge_loss


# speedup vs baseline: 1.9591x; 1.0130x over previous
import jax
import jax.numpy as jnp
from jax import lax
from jax.experimental import pallas as pl
from jax.experimental.pallas import tpu as pltpu

_VMEM_LIMIT = 32 * 1024 * 1024
_NODE_TILE = 4096
_TILE_R = 256
_GUNROLL = 32
_LANE = 128


def _cdiv(a, b):
    return -(-a // b)


def _round_up(v, m):
    return ((v + m - 1) // m) * m


def _node_kernel(x_ref, wdT_ref, wsd_ref, abT_ref):
    w2t = jnp.dot(wdT_ref[...], wsd_ref[...], preferred_element_type=jnp.float32)
    abT_ref[...] = lax.dot_general(
        w2t, x_ref[...], (((0,), (1,)), ((), ())),
        preferred_element_type=jnp.float32)


def _edge_kernel(hiA_ref, hiB_ref, loA_ref, loB_ref, t_ref, a_ref, b_ref,
                 score_ref, lsum_ref, msum_ref,
                 hiA_s, hiB_s, *scratch):
    sems = scratch[-1]
    scratch = scratch[:-1]
    cpa = pltpu.make_async_copy(hiA_ref, hiA_s, sems.at[0])
    cpb = pltpu.make_async_copy(hiB_ref, hiB_s, sems.at[1])
    cpa.start()
    cpb.start()
    cpa.wait()
    cpb.wait()

    rtas = scratch[0:2 * _GUNROLL:2]
    rtbs = scratch[1:2 * _GUNROLL:2]
    iota_l = lax.broadcasted_iota(jnp.int32, (_LANE, _LANE), 1)
    ones_col = jnp.ones((_LANE, 1), jnp.float32)

    def body(q, carry):
        lacc, macc = carry
        g0 = q * _GUNROLL
        for u in range(_GUNROLL):
            for mi in range(_LANE):
                rtas[u][pl.ds(mi, 1), :] = a_ref[hiA_s[g0 + u, mi]]
                rtbs[u][pl.ds(mi, 1), :] = b_ref[hiB_s[g0 + u, mi]]
        zcs = []
        for u in range(_GUNROLL):
            g = g0 + u
            lac = loA_ref[g].reshape(_LANE, 1)
            lbc = loB_ref[g].reshape(_LANE, 1)
            m = (jnp.where(iota_l == lac, rtas[u][...], 0.0)
                 + jnp.where(iota_l == lbc, rtbs[u][...], 0.0))
            zcs.append(jnp.dot(m, ones_col, preferred_element_type=jnp.float32))
        z = jnp.concatenate(zcs, axis=1).T
        score = 0.5 * (jnp.tanh(0.5 * z) + 1.0)
        score_ref[q] = score
        p = 0.5 * (jnp.tanh(0.5 * score) + 1.0)
        t = t_ref[q]
        w = jnp.abs(t)
        pt = jnp.where(t > 0.0, p, 1.0 - p)
        om = 1.0 - pt
        lacc = lacc - (om * om) * jnp.log(pt) * w
        macc = macc + jnp.where(t > 0.0, w * (1.0 / 0.75), w * (1.0 / 0.25))
        return lacc, macc

    lacc, macc = lax.fori_loop(
        0, _TILE_R // _GUNROLL, body,
        (jnp.zeros((_GUNROLL, _LANE), jnp.float32),
         jnp.zeros((_GUNROLL, _LANE), jnp.float32)))
    lsum_ref[...] = jnp.sum(lacc, axis=0, keepdims=True)[None]
    msum_ref[...] = jnp.sum(macc, axis=0, keepdims=True)[None]


def kernel(d_liner_w, f_liner_w, x, edge_index, edge_labels, edge_train_mask):
    alpha = 0.25
    wd = d_liner_w.astype(jnp.float32)
    wf = f_liner_w.astype(jnp.float32)
    c_out, c_in = wd.shape
    w1 = wf[0, 0 * c_out:1 * c_out]
    w2 = wf[0, 1 * c_out:2 * c_out]
    w3 = wf[0, 2 * c_out:3 * c_out]
    wsd = jnp.stack([w1 + w3, w2 - w3], axis=1)
    wdT = wd.T

    n = x.shape[0]
    n_pad = _round_up(n, _NODE_TILE)
    xp = jnp.pad(x.astype(jnp.float32), ((0, n_pad - n), (0, 0)))
    abT = pl.pallas_call(
        _node_kernel,
        out_shape=jax.ShapeDtypeStruct((2, n_pad), jnp.float32),
        grid=(n_pad // _NODE_TILE,),
        in_specs=[pl.BlockSpec((_NODE_TILE, c_in), lambda i: (i, 0)),
                  pl.BlockSpec((c_in, c_out), lambda i: (0, 0)),
                  pl.BlockSpec((c_out, 2), lambda i: (0, 0))],
        out_specs=pl.BlockSpec((2, _NODE_TILE), lambda i: (0, i)),
        compiler_params=pltpu.CompilerParams(
            dimension_semantics=("parallel",),
            vmem_limit_bytes=_VMEM_LIMIT),
        cost_estimate=pl.CostEstimate(
            flops=2 * n_pad * c_in * 2 + 2 * c_in * c_out * 2,
            transcendentals=0,
            bytes_accessed=4 * (n_pad * c_in + c_in * c_out + 2 * c_out
                                + 2 * n_pad)),
    )(xp, wdT, wsd)

    n_rows = n_pad // _LANE
    a_tab = abT[0].reshape(n_rows, 1, _LANE)
    b_tab = abT[1].reshape(n_rows, 1, _LANE)

    e = edge_index.shape[1]
    r_pad = _round_up(_cdiv(e, _LANE), _TILE_R)
    e_pad = r_pad * _LANE
    grid_sz = r_pad // _TILE_R

    ei0 = jnp.pad(edge_index[0], (0, e_pad - e))
    ei1 = jnp.pad(edge_index[1], (0, e_pad - e))
    hiA = (ei0 >> 7).reshape(r_pad, _LANE)
    hiB = (ei1 >> 7).reshape(r_pad, _LANE)
    loA = (ei0 & 127).reshape(r_pad, 1, _LANE)
    loB = (ei1 & 127).reshape(r_pad, 1, _LANE)

    lab = edge_labels.astype(jnp.float32)
    mask = edge_train_mask.astype(jnp.float32)
    at = jnp.where(lab > 0.5, 1.0 - alpha, alpha)
    t = jnp.where(lab > 0.5, at, -at) * mask
    t3 = jnp.pad(t, (0, e_pad - e)).reshape(r_pad // _GUNROLL, _GUNROLL, _LANE)

    score3, lparts, mparts = pl.pallas_call(
        _edge_kernel,
        out_shape=(jax.ShapeDtypeStruct((r_pad // _GUNROLL, _GUNROLL, _LANE),
                                        jnp.float32),
                   jax.ShapeDtypeStruct((grid_sz, 1, _LANE), jnp.float32),
                   jax.ShapeDtypeStruct((grid_sz, 1, _LANE), jnp.float32)),
        grid=(grid_sz,),
        in_specs=[pl.BlockSpec((_TILE_R, _LANE), lambda i: (i, 0)),
                  pl.BlockSpec((_TILE_R, _LANE), lambda i: (i, 0)),
                  pl.BlockSpec((_TILE_R, 1, _LANE), lambda i: (i, 0, 0)),
                  pl.BlockSpec((_TILE_R, 1, _LANE), lambda i: (i, 0, 0)),
                  pl.BlockSpec((_TILE_R // _GUNROLL, _GUNROLL, _LANE),
                               lambda i: (i, 0, 0)),
                  pl.BlockSpec((n_rows, 1, _LANE), lambda i: (0, 0, 0)),
                  pl.BlockSpec((n_rows, 1, _LANE), lambda i: (0, 0, 0))],
        out_specs=(pl.BlockSpec((_TILE_R // _GUNROLL, _GUNROLL, _LANE),
                                lambda i: (i, 0, 0)),
                   pl.BlockSpec((1, 1, _LANE), lambda i: (i, 0, 0)),
                   pl.BlockSpec((1, 1, _LANE), lambda i: (i, 0, 0))),
        scratch_shapes=[pltpu.SMEM((_TILE_R, _LANE), jnp.int32),
                        pltpu.SMEM((_TILE_R, _LANE), jnp.int32)]
                       + [pltpu.VMEM((_LANE, _LANE), jnp.float32)] * (2 * _GUNROLL)
                       + [pltpu.SemaphoreType.DMA((2,))],
        compiler_params=pltpu.CompilerParams(
            dimension_semantics=("parallel",),
            vmem_limit_bytes=_VMEM_LIMIT),
        cost_estimate=pl.CostEstimate(
            flops=2 * e_pad * _LANE + 12 * e_pad,
            transcendentals=3 * e_pad,
            bytes_accessed=4 * (e_pad * 4 + 2 * n_pad + e_pad)),
    )(hiA, hiB, loA, loB, t3, a_tab, b_tab)

    edge_score = score3.reshape(-1)[:e]
    edge_loss = jnp.sum(lparts) / jnp.sum(mparts)
    return edge_score, edge_loss


# speedup vs baseline: 1.9628x; 1.0019x over previous
import jax
import jax.numpy as jnp
from jax import lax
from jax.experimental import pallas as pl
from jax.experimental.pallas import tpu as pltpu

_VMEM_LIMIT = 32 * 1024 * 1024
_NODE_TILE = 8192
_TILE_R = 512
_GUNROLL = 32
_LANE = 128


def _cdiv(a, b):
    return -(-a // b)


def _round_up(v, m):
    return ((v + m - 1) // m) * m


def _node_kernel(x_ref, wdT_ref, wsd_ref, abT_ref):
    w2t = jnp.dot(wdT_ref[...], wsd_ref[...], preferred_element_type=jnp.float32)
    abT_ref[...] = lax.dot_general(
        w2t, x_ref[...], (((0,), (1,)), ((), ())),
        preferred_element_type=jnp.float32)


def _edge_kernel(hiA_ref, hiB_ref, loA_ref, loB_ref, t_ref, a_ref, b_ref,
                 score_ref, lsum_ref, msum_ref,
                 hiA_s, hiB_s, *scratch):
    sems = scratch[-1]
    scratch = scratch[:-1]
    cpa = pltpu.make_async_copy(hiA_ref, hiA_s, sems.at[0])
    cpb = pltpu.make_async_copy(hiB_ref, hiB_s, sems.at[1])
    cpa.start()
    cpb.start()
    cpa.wait()
    cpb.wait()

    rtas = scratch[0:2 * _GUNROLL:2]
    rtbs = scratch[1:2 * _GUNROLL:2]
    iota_l = lax.broadcasted_iota(jnp.int32, (_LANE, _LANE), 1)
    ones_col = jnp.ones((_LANE, 1), jnp.float32)

    def body(q, carry):
        lacc, macc = carry
        g0 = q * _GUNROLL
        for u in range(_GUNROLL):
            for mi in range(_LANE):
                rtas[u][pl.ds(mi, 1), :] = a_ref[hiA_s[g0 + u, mi]]
                rtbs[u][pl.ds(mi, 1), :] = b_ref[hiB_s[g0 + u, mi]]
        zcs = []
        for u in range(_GUNROLL):
            g = g0 + u
            lac = loA_ref[g].reshape(_LANE, 1)
            lbc = loB_ref[g].reshape(_LANE, 1)
            m = (jnp.where(iota_l == lac, rtas[u][...], 0.0)
                 + jnp.where(iota_l == lbc, rtbs[u][...], 0.0))
            zcs.append(jnp.dot(m, ones_col, preferred_element_type=jnp.float32))
        z = jnp.concatenate(zcs, axis=1).T
        score = 0.5 * (jnp.tanh(0.5 * z) + 1.0)
        score_ref[q] = score
        p = 0.5 * (jnp.tanh(0.5 * score) + 1.0)
        t = t_ref[q]
        w = jnp.abs(t)
        pt = jnp.where(t > 0.0, p, 1.0 - p)
        om = 1.0 - pt
        lacc = lacc - (om * om) * jnp.log(pt) * w
        macc = macc + jnp.where(t > 0.0, w * (1.0 / 0.75), w * (1.0 / 0.25))
        return lacc, macc

    lacc, macc = lax.fori_loop(
        0, _TILE_R // _GUNROLL, body,
        (jnp.zeros((_GUNROLL, _LANE), jnp.float32),
         jnp.zeros((_GUNROLL, _LANE), jnp.float32)))
    lsum_ref[...] = jnp.sum(lacc, axis=0, keepdims=True)[None]
    msum_ref[...] = jnp.sum(macc, axis=0, keepdims=True)[None]


def kernel(d_liner_w, f_liner_w, x, edge_index, edge_labels, edge_train_mask):
    alpha = 0.25
    wd = d_liner_w.astype(jnp.float32)
    wf = f_liner_w.astype(jnp.float32)
    c_out, c_in = wd.shape
    w1 = wf[0, 0 * c_out:1 * c_out]
    w2 = wf[0, 1 * c_out:2 * c_out]
    w3 = wf[0, 2 * c_out:3 * c_out]
    wsd = jnp.stack([w1 + w3, w2 - w3], axis=1)
    wdT = wd.T

    n = x.shape[0]
    n_pad = _round_up(n, _NODE_TILE)
    xp = jnp.pad(x.astype(jnp.float32), ((0, n_pad - n), (0, 0)))
    abT = pl.pallas_call(
        _node_kernel,
        out_shape=jax.ShapeDtypeStruct((2, n_pad), jnp.float32),
        grid=(n_pad // _NODE_TILE,),
        in_specs=[pl.BlockSpec((_NODE_TILE, c_in), lambda i: (i, 0)),
                  pl.BlockSpec((c_in, c_out), lambda i: (0, 0)),
                  pl.BlockSpec((c_out, 2), lambda i: (0, 0))],
        out_specs=pl.BlockSpec((2, _NODE_TILE), lambda i: (0, i)),
        compiler_params=pltpu.CompilerParams(
            dimension_semantics=("parallel",),
            vmem_limit_bytes=_VMEM_LIMIT),
        cost_estimate=pl.CostEstimate(
            flops=2 * n_pad * c_in * 2 + 2 * c_in * c_out * 2,
            transcendentals=0,
            bytes_accessed=4 * (n_pad * c_in + c_in * c_out + 2 * c_out
                                + 2 * n_pad)),
    )(xp, wdT, wsd)

    n_rows = n_pad // _LANE
    a_tab = abT[0].reshape(n_rows, 1, _LANE)
    b_tab = abT[1].reshape(n_rows, 1, _LANE)

    e = edge_index.shape[1]
    r_pad = _round_up(_cdiv(e, _LANE), _TILE_R)
    e_pad = r_pad * _LANE
    grid_sz = r_pad // _TILE_R

    ei0 = jnp.pad(edge_index[0], (0, e_pad - e))
    ei1 = jnp.pad(edge_index[1], (0, e_pad - e))
    hiA = (ei0 >> 7).reshape(r_pad, _LANE)
    hiB = (ei1 >> 7).reshape(r_pad, _LANE)
    loA = (ei0 & 127).reshape(r_pad, 1, _LANE)
    loB = (ei1 & 127).reshape(r_pad, 1, _LANE)

    lab = edge_labels.astype(jnp.float32)
    mask = edge_train_mask.astype(jnp.float32)
    at = jnp.where(lab > 0.5, 1.0 - alpha, alpha)
    t = jnp.where(lab > 0.5, at, -at) * mask
    t3 = jnp.pad(t, (0, e_pad - e)).reshape(r_pad // _GUNROLL, _GUNROLL, _LANE)

    score3, lparts, mparts = pl.pallas_call(
        _edge_kernel,
        out_shape=(jax.ShapeDtypeStruct((r_pad // _GUNROLL, _GUNROLL, _LANE),
                                        jnp.float32),
                   jax.ShapeDtypeStruct((grid_sz, 1, _LANE), jnp.float32),
                   jax.ShapeDtypeStruct((grid_sz, 1, _LANE), jnp.float32)),
        grid=(grid_sz,),
        in_specs=[pl.BlockSpec((_TILE_R, _LANE), lambda i: (i, 0)),
                  pl.BlockSpec((_TILE_R, _LANE), lambda i: (i, 0)),
                  pl.BlockSpec((_TILE_R, 1, _LANE), lambda i: (i, 0, 0)),
                  pl.BlockSpec((_TILE_R, 1, _LANE), lambda i: (i, 0, 0)),
                  pl.BlockSpec((_TILE_R // _GUNROLL, _GUNROLL, _LANE),
                               lambda i: (i, 0, 0)),
                  pl.BlockSpec((n_rows, 1, _LANE), lambda i: (0, 0, 0)),
                  pl.BlockSpec((n_rows, 1, _LANE), lambda i: (0, 0, 0))],
        out_specs=(pl.BlockSpec((_TILE_R // _GUNROLL, _GUNROLL, _LANE),
                                lambda i: (i, 0, 0)),
                   pl.BlockSpec((1, 1, _LANE), lambda i: (i, 0, 0)),
                   pl.BlockSpec((1, 1, _LANE), lambda i: (i, 0, 0))),
        scratch_shapes=[pltpu.SMEM((_TILE_R, _LANE), jnp.int32),
                        pltpu.SMEM((_TILE_R, _LANE), jnp.int32)]
                       + [pltpu.VMEM((_LANE, _LANE), jnp.float32)] * (2 * _GUNROLL)
                       + [pltpu.SemaphoreType.DMA((2,))],
        compiler_params=pltpu.CompilerParams(
            dimension_semantics=("parallel",),
            vmem_limit_bytes=_VMEM_LIMIT),
        cost_estimate=pl.CostEstimate(
            flops=2 * e_pad * _LANE + 12 * e_pad,
            transcendentals=3 * e_pad,
            bytes_accessed=4 * (e_pad * 4 + 2 * n_pad + e_pad)),
    )(hiA, hiB, loA, loB, t3, a_tab, b_tab)

    edge_score = score3.reshape(-1)[:e]
    edge_loss = jnp.sum(lparts) / jnp.sum(mparts)
    return edge_score, edge_loss
